# baseline (speedup 1.0000x reference)
import functools

import jax
import jax.numpy as jnp
from jax import lax
from jax.experimental import pallas as pl
from jax.experimental.pallas import tpu as pltpu
from jax.experimental.pallas import tpu_sc as plsc

N = 10000
D = 128
NC = 2
NS = 16
NW = NC * NS
L = 16

E_PAD = 163840
EPT = E_PAD // NW
CHW = 128
CH = EPT // CHW
NPAD = 10240
ROWS_PT = NPAD // NS

_mesh = plsc.VectorSubcoreMesh(
    core_axis_name="c", subcore_axis_name="s", num_cores=NC, num_subcores=NS
)

_sc_params = pltpu.CompilerParams(needs_layout_passes=False)


@functools.partial(
    pl.kernel,
    out_type=jax.ShapeDtypeStruct((NW, N), jnp.float32),
    mesh=_mesh,
    scratch_types=[
        pltpu.VMEM((EPT,), jnp.int32),
        pltpu.VMEM((EPT,), jnp.float32),
        pltpu.VMEM((N,), jnp.float32),
    ],
    compiler_params=_sc_params,
)
def _deg_kernel(rows_hbm, w_hbm, deg_out, rows_v, w_v, deg_v):
    wid = lax.axis_index("c") * NS + lax.axis_index("s")
    pltpu.sync_copy(rows_hbm.at[pl.ds(wid * EPT, EPT)], rows_v)
    pltpu.sync_copy(w_hbm.at[pl.ds(wid * EPT, EPT)], w_v)

    z16 = jnp.zeros((L,), jnp.float32)

    def zero_body(i, carry):
        deg_v[pl.ds(i * L, L)] = z16
        return carry

    lax.fori_loop(0, N // L, zero_body, 0, unroll=8)

    def grp(gi, carry):
        r = rows_v[pl.ds(gi * L, L)]
        wv = w_v[pl.ds(gi * L, L)]
        plsc.addupdate_scatter(deg_v, [r], wv)
        return carry

    lax.fori_loop(0, EPT // L, grp, 0, unroll=4)

    pltpu.sync_copy(deg_v, deg_out.at[wid])


@functools.partial(
    pl.kernel,
    out_type=jax.ShapeDtypeStruct((NC, NPAD, D), jnp.float32),
    mesh=_mesh,
    scratch_types=[
        pltpu.VMEM((CH, CHW), jnp.int32),
        pltpu.VMEM((CH, CHW), jnp.int32),
        pltpu.VMEM((EPT,), jnp.float32),
        pltpu.VMEM((CHW, D), jnp.float32),
        pltpu.VMEM((CHW, D), jnp.float32),
        pltpu.SemaphoreType.DMA,
        pltpu.SemaphoreType.DMA,
        pltpu.VMEM_SHARED((NPAD, D), jnp.float32),
    ],
    compiler_params=_sc_params,
)
def _agg_kernel(g_hbm, cols_hbm, rows_hbm, w_hbm, out_hbm,
                cols_v, rows_v, w_v, buf0, buf1, sem0, sem1, acc):
    ci = lax.axis_index("c")
    si = lax.axis_index("s")
    wid = ci * NS + si
    pltpu.sync_copy(cols_hbm.at[pl.ds(wid * CH, CH)], cols_v)
    pltpu.sync_copy(rows_hbm.at[pl.ds(wid * CH, CH)], rows_v)
    pltpu.sync_copy(w_hbm.at[pl.ds(wid * EPT, EPT)], w_v)

    z16 = jnp.zeros((L,), jnp.float32)

    def zb(i, carry):
        for k in range(D // L):
            buf0[i, pl.ds(k * L, L)] = z16
        return carry

    lax.fori_loop(0, CHW, zb, 0, unroll=8)
    base = si * ROWS_PT
    for i in range(ROWS_PT // CHW):
        pltpu.sync_copy(buf0, acc.at[pl.ds(base + i * CHW, CHW)])
    plsc.subcore_barrier()

    def scale(buf, c):
        ebase = c * CHW

        def edge(j, icarry):
            idx = jnp.full((L,), ebase + j, jnp.int32)
            wsp = plsc.load_gather(w_v, [idx])
            row = buf.at[j]
            for k in range(D // L):
                row[pl.ds(k * L, L)] = row[pl.ds(k * L, L)] * wsp
            return icarry

        lax.fori_loop(0, CHW, edge, 0, unroll=8)

    def wait_gather(buf, sem):
        pltpu.make_async_copy(g_hbm.at[pl.ds(0, CHW)], buf, sem).wait()

    pltpu.async_copy(g_hbm.at[cols_v.at[0]], buf0, sem0)

    def pair(t, carry):
        c0 = 2 * t
        c1 = 2 * t + 1
        wait_gather(buf0, sem0)
        pltpu.async_copy(g_hbm.at[cols_v.at[c1]], buf1, sem1)
        scale(buf0, c0)
        pltpu.sync_copy(buf0, acc.at[rows_v.at[c0]], add=True)
        wait_gather(buf1, sem1)

        @pl.when(t + 1 < CH // 2)
        def _():
            pltpu.async_copy(g_hbm.at[cols_v.at[c0 + 2]], buf0, sem0)

        scale(buf1, c1)
        pltpu.sync_copy(buf1, acc.at[rows_v.at[c1]], add=True)
        return carry

    lax.fori_loop(0, CH // 2, pair, 0)

    plsc.subcore_barrier()
    pltpu.sync_copy(acc.at[pl.ds(base, ROWS_PT)],
                    out_hbm.at[ci, pl.ds(base, ROWS_PT)])


BLK = 1000
GRID = N // BLK


def _mm_body(x_ref, w_ref, h_ref):
    h_ref[...] = jnp.dot(x_ref[...], w_ref[...],
                         preferred_element_type=jnp.float32)


def _mm_call(x, W):
    return pl.pallas_call(
        _mm_body,
        grid=(GRID,),
        in_specs=[
            pl.BlockSpec((BLK, D), lambda i: (i, 0)),
            pl.BlockSpec((D, D), lambda i: (0, 0)),
        ],
        out_specs=pl.BlockSpec((BLK, D), lambda i: (i, 0)),
        out_shape=jax.ShapeDtypeStruct((N, D), jnp.float32),
    )(x, W)


def _scale_body(h_ref, degp_ref, g_ref, dinv_ref):
    deg = jnp.sum(degp_ref[...], axis=0)
    safe = jnp.where(deg > 0, deg, 1.0)
    dinv = jnp.where(deg > 0, lax.rsqrt(safe), 0.0)
    g_ref[...] = h_ref[...] * dinv[:, None]
    dinv_ref[...] = dinv[:, None]


def _scale_call(h, deg_parts):
    return pl.pallas_call(
        _scale_body,
        out_shape=[
            jax.ShapeDtypeStruct((N, D), jnp.float32),
            jax.ShapeDtypeStruct((N, 1), jnp.float32),
        ],
    )(h, deg_parts)


def _post_body(p_ref, dinv_ref, o_ref):
    s = p_ref[0] + p_ref[1]
    o_ref[...] = jnp.maximum(s * dinv_ref[...], 0.0)


def _post_call(parts, dinv):
    return pl.pallas_call(
        _post_body,
        grid=(GRID,),
        in_specs=[
            pl.BlockSpec((NC, BLK, D), lambda i: (0, i, 0)),
            pl.BlockSpec((BLK, 1), lambda i: (i, 0)),
        ],
        out_specs=pl.BlockSpec((BLK, D), lambda i: (i, 0)),
        out_shape=jax.ShapeDtypeStruct((N, D), jnp.float32),
    )(parts, dinv)


def kernel(x, u_edge_index, u_edge_weight, v_edge_index, v_edge_weight, W):
    x = x.astype(jnp.float32)
    W = W.astype(jnp.float32)
    ei = jnp.concatenate([u_edge_index, v_edge_index], axis=1).astype(jnp.int32)
    ew = jnp.concatenate([u_edge_weight, v_edge_weight], axis=0)
    ew = ew.astype(jnp.float32)
    e = ei.shape[1]
    pad = E_PAD - e
    rows = jnp.concatenate([ei[1], jnp.zeros((pad,), jnp.int32)])
    cols = jnp.concatenate([ei[0], jnp.zeros((pad,), jnp.int32)])
    w = jnp.concatenate([ew, jnp.zeros((pad,), jnp.float32)])
    cols2d = cols.reshape(NW * CH, CHW)
    rows2d = rows.reshape(NW * CH, CHW)

    deg_parts = _deg_kernel(rows, w)
    h = _mm_call(x, W)
    g, dinv = _scale_call(h, deg_parts)
    parts = _agg_kernel(g, cols2d, rows2d, w)
    return _post_call(parts, dinv)


# speedup vs baseline: 1.0230x; 1.0230x over previous
import functools

import jax
import jax.numpy as jnp
from jax import lax
from jax.experimental import pallas as pl
from jax.experimental.pallas import tpu as pltpu
from jax.experimental.pallas import tpu_sc as plsc

N = 10000
D = 128
NC = 2
NS = 16
NW = NC * NS
L = 16

E_PAD = 163840
EPT = E_PAD // NW
CHW = 128
CH = EPT // CHW
NPAD = 10240
ROWS_PT = NPAD // NS

_mesh = plsc.VectorSubcoreMesh(
    core_axis_name="c", subcore_axis_name="s", num_cores=NC, num_subcores=NS
)

_sc_params = pltpu.CompilerParams(needs_layout_passes=False)


@functools.partial(
    pl.kernel,
    out_type=jax.ShapeDtypeStruct((NW, N), jnp.float32),
    mesh=_mesh,
    scratch_types=[
        pltpu.VMEM((EPT,), jnp.int32),
        pltpu.VMEM((EPT,), jnp.float32),
        pltpu.VMEM((N,), jnp.float32),
    ],
    compiler_params=_sc_params,
)
def _deg_kernel(rows_hbm, w_hbm, deg_out, rows_v, w_v, deg_v):
    wid = lax.axis_index("c") * NS + lax.axis_index("s")
    pltpu.sync_copy(rows_hbm.at[pl.ds(wid * EPT, EPT)], rows_v)
    pltpu.sync_copy(w_hbm.at[pl.ds(wid * EPT, EPT)], w_v)

    z16 = jnp.zeros((L,), jnp.float32)

    def zero_body(i, carry):
        deg_v[pl.ds(i * L, L)] = z16
        return carry

    lax.fori_loop(0, N // L, zero_body, 0, unroll=8)

    def grp(gi, carry):
        r = rows_v[pl.ds(gi * L, L)]
        wv = w_v[pl.ds(gi * L, L)]
        plsc.addupdate_scatter(deg_v, [r], wv)
        return carry

    lax.fori_loop(0, EPT // L, grp, 0, unroll=4)

    pltpu.sync_copy(deg_v, deg_out.at[wid])


CH_FAST = 56
CH_SLOW = NC * CH - CH_FAST
FAST_CORE = 0
CHUNK_OFF = NS * CH_FAST
CH_ALLOC = NC * NS * CH
E_ALLOC = CH_ALLOC * CHW
NSLOT = 2


@functools.partial(
    pl.kernel,
    out_type=jax.ShapeDtypeStruct((NC, NPAD, D), jnp.float32),
    mesh=_mesh,
    scratch_types=[
        pltpu.VMEM((NSLOT, CHW), jnp.int32),
        pltpu.VMEM((NSLOT, CHW), jnp.int32),
        pltpu.VMEM((NSLOT, CHW), jnp.float32),
        [pltpu.VMEM((CHW, D), jnp.float32) for _ in range(NSLOT)],
        [pltpu.SemaphoreType.DMA for _ in range(NSLOT)],
        [pltpu.SemaphoreType.DMA for _ in range(NSLOT)],
        pltpu.VMEM_SHARED((NPAD, D), jnp.float32),
    ],
    compiler_params=_sc_params,
)
def _agg_kernel(g_hbm, cols_hbm, rows_hbm, w_hbm, out_hbm,
                cols_v, rows_v, w_v, bufs, gsems, isems, acc):
    ci = lax.axis_index("c")
    si = lax.axis_index("s")

    z16 = jnp.zeros((L,), jnp.float32)
    buf0 = bufs[0]

    def zb(i, carry):
        for k in range(D // L):
            buf0[i, pl.ds(k * L, L)] = z16
        return carry

    lax.fori_loop(0, CHW, zb, 0, unroll=8)
    base = si * ROWS_PT
    for i in range(ROWS_PT // CHW):
        pltpu.sync_copy(buf0, acc.at[pl.ds(base + i * CHW, CHW)])
    plsc.subcore_barrier()

    is_fast = ci == FAST_CORE
    nch = jnp.where(is_fast, CH_FAST, CH_SLOW)
    start = jnp.where(is_fast, si * CH_FAST, CHUNK_OFF + si * CH_SLOW)

    def prefetch_idx(s, gc):
        pltpu.async_copy(cols_hbm.at[gc], cols_v.at[s], isems[s])
        pltpu.async_copy(rows_hbm.at[gc], rows_v.at[s], isems[s])
        pltpu.async_copy(w_hbm.at[gc], w_v.at[s], isems[s])

    def wait_idx(s):
        pltpu.make_async_copy(cols_hbm.at[0], cols_v.at[s], isems[s]).wait()
        pltpu.make_async_copy(rows_hbm.at[0], rows_v.at[s], isems[s]).wait()
        pltpu.make_async_copy(w_hbm.at[0], w_v.at[s], isems[s]).wait()

    def start_gather(s):
        pltpu.async_copy(g_hbm.at[cols_v.at[s]], bufs[s], gsems[s])

    def wait_gather(s):
        pltpu.make_async_copy(g_hbm.at[pl.ds(0, CHW)], bufs[s],
                              gsems[s]).wait()

    def scale(s):
        wrow = w_v.at[s]
        buf = bufs[s]

        def edge(j, icarry):
            idx = jnp.full((L,), j, jnp.int32)
            wsp = plsc.load_gather(wrow, [idx])
            row = buf.at[j]
            for k in range(D // L):
                row[pl.ds(k * L, L)] = row[pl.ds(k * L, L)] * wsp
            return icarry

        lax.fori_loop(0, CHW, edge, 0, unroll=8)

    prefetch_idx(0, start)
    prefetch_idx(1, start + 1)
    wait_idx(0)
    start_gather(0)

    def pair(t, carry):
        c0 = 2 * t
        wait_gather(0)
        wait_idx(1)
        start_gather(1)
        scale(0)
        pltpu.sync_copy(bufs[0], acc.at[rows_v.at[0]], add=True)

        @pl.when(c0 + 2 < nch)
        def _():
            prefetch_idx(0, start + c0 + 2)

        wait_gather(1)

        @pl.when(c0 + 2 < nch)
        def _():
            wait_idx(0)
            start_gather(0)

        scale(1)
        pltpu.sync_copy(bufs[1], acc.at[rows_v.at[1]], add=True)

        @pl.when(c0 + 3 < nch)
        def _():
            prefetch_idx(1, start + c0 + 3)

        return carry

    lax.fori_loop(0, nch // 2, pair, 0)

    plsc.subcore_barrier()
    pltpu.sync_copy(acc.at[pl.ds(base, ROWS_PT)],
                    out_hbm.at[ci, pl.ds(base, ROWS_PT)])


BLK = 1000
GRID = N // BLK


def _mm_body(x_ref, w_ref, h_ref):
    h_ref[...] = jnp.dot(x_ref[...], w_ref[...],
                         preferred_element_type=jnp.float32)


def _mm_call(x, W):
    return pl.pallas_call(
        _mm_body,
        grid=(GRID,),
        in_specs=[
            pl.BlockSpec((BLK, D), lambda i: (i, 0)),
            pl.BlockSpec((D, D), lambda i: (0, 0)),
        ],
        out_specs=pl.BlockSpec((BLK, D), lambda i: (i, 0)),
        out_shape=jax.ShapeDtypeStruct((N, D), jnp.float32),
    )(x, W)


def _scale_body(h_ref, degp_ref, g_ref, dinv_ref):
    deg = jnp.sum(degp_ref[...], axis=0)
    safe = jnp.where(deg > 0, deg, 1.0)
    dinv = jnp.where(deg > 0, lax.rsqrt(safe), 0.0)
    g_ref[...] = h_ref[...] * dinv[:, None]
    dinv_ref[...] = dinv[:, None]


def _scale_call(h, deg_parts):
    return pl.pallas_call(
        _scale_body,
        out_shape=[
            jax.ShapeDtypeStruct((N, D), jnp.float32),
            jax.ShapeDtypeStruct((N, 1), jnp.float32),
        ],
    )(h, deg_parts)


def _post_body(p_ref, dinv_ref, o_ref):
    s = p_ref[0] + p_ref[1]
    o_ref[...] = jnp.maximum(s * dinv_ref[...], 0.0)


def _post_call(parts, dinv):
    return pl.pallas_call(
        _post_body,
        grid=(GRID,),
        in_specs=[
            pl.BlockSpec((NC, BLK, D), lambda i: (0, i, 0)),
            pl.BlockSpec((BLK, 1), lambda i: (i, 0)),
        ],
        out_specs=pl.BlockSpec((BLK, D), lambda i: (i, 0)),
        out_shape=jax.ShapeDtypeStruct((N, D), jnp.float32),
    )(parts, dinv)


def kernel(x, u_edge_index, u_edge_weight, v_edge_index, v_edge_weight, W):
    x = x.astype(jnp.float32)
    W = W.astype(jnp.float32)
    ei = jnp.concatenate([u_edge_index, v_edge_index], axis=1).astype(jnp.int32)
    ew = jnp.concatenate([u_edge_weight, v_edge_weight], axis=0)
    ew = ew.astype(jnp.float32)
    e = ei.shape[1]
    pad = E_ALLOC - e
    rows = jnp.concatenate([ei[1], jnp.zeros((pad,), jnp.int32)])
    cols = jnp.concatenate([ei[0], jnp.zeros((pad,), jnp.int32)])
    w = jnp.concatenate([ew, jnp.zeros((pad,), jnp.float32)])
    cols2d = cols.reshape(CH_ALLOC, CHW)
    rows2d = rows.reshape(CH_ALLOC, CHW)
    w2d = w.reshape(CH_ALLOC, CHW)

    deg_parts = _deg_kernel(rows, w)
    h = _mm_call(x, W)
    g, dinv = _scale_call(h, deg_parts)
    parts = _agg_kernel(g, cols2d, rows2d, w2d)
    return _post_call(parts, dinv)
